# baseline (speedup 1.0000x reference)
import jax
import jax.numpy as jnp
from jax.experimental import pallas as pl
from jax.experimental.pallas import tpu as pltpu

_IOU_T = 0.5
_SCORE_T = 0.8
_MAXDET = 300
_OUTW = 384
_B = 4
_NPAD = 5120
_CHUNK = 128
_NCH = _NPAD // _CHUNK
_ROWS = _B * _NCH


def _nms_kernel(scores_ref, x1_ref, y1_ref, x2_ref, y2_ref, cls_ref,
                out_s_ref, out_b_ref, out_c_ref, out_n_ref,
                work_ref, cm_ref, kb_ref, ka_ref):
    lane = jax.lax.broadcasted_iota(jnp.int32, (1, _CHUNK), 1)
    slot = jax.lax.broadcasted_iota(jnp.int32, (1, _OUTW), 1)
    riota = jax.lax.broadcasted_iota(jnp.int32, (_NCH, 1), 0)

    work_ref[...] = scores_ref[...]
    cm_ref[...] = jnp.max(scores_ref[...], axis=1, keepdims=True)
    out_s_ref[...] = jnp.zeros_like(out_s_ref)
    out_b_ref[...] = jnp.zeros_like(out_b_ref)
    out_c_ref[...] = jnp.zeros_like(out_c_ref)
    kb_ref[...] = jnp.zeros_like(kb_ref)
    ka_ref[...] = jnp.zeros_like(ka_ref)

    valid = scores_ref[...] > _SCORE_T
    mc = jnp.float32(-jnp.inf)
    for pref in (x1_ref, y1_ref, x2_ref, y2_ref):
        mc = jnp.maximum(mc, jnp.max(jnp.where(valid, pref[...], -jnp.inf)))
    mc = mc + 1.0

    def iter_body(carry):
        t = carry[0]
        ds = list(carry[1:5])
        cs = list(carry[5:9])
        for b in range(_B):
            cmb = cm_ref[b * _NCH:(b + 1) * _NCH, 0:1]
            m = jnp.max(cmb)
            c = jnp.min(jnp.where(cmb == m, riota, _NCH))
            r = b * _NCH + c
            chunk = work_ref[pl.ds(r, 1), :]
            lidx = jnp.min(jnp.where(chunk == m, lane, _CHUNK))
            oh = lane == lidx
            act = (m > _SCORE_T) & jnp.logical_not(ds[b])
            off = jnp.float32(b) * mc
            gx1 = jnp.sum(jnp.where(oh, x1_ref[pl.ds(r, 1), :], 0.0))
            gy1 = jnp.sum(jnp.where(oh, y1_ref[pl.ds(r, 1), :], 0.0))
            gx2 = jnp.sum(jnp.where(oh, x2_ref[pl.ds(r, 1), :], 0.0))
            gy2 = jnp.sum(jnp.where(oh, y2_ref[pl.ds(r, 1), :], 0.0))
            ccls = jnp.sum(jnp.where(oh, cls_ref[pl.ds(r, 1), :], 0))
            cx1 = gx1 + off
            cy1 = gy1 + off
            cx2 = gx2 + off
            cy2 = gy2 + off
            carea = jnp.maximum(cx2 - cx1, 0.0) * jnp.maximum(cy2 - cy1, 0.0)
            kx1 = kb_ref[b, 0:1, :]
            ky1 = kb_ref[b, 1:2, :]
            kx2 = kb_ref[b, 2:3, :]
            ky2 = kb_ref[b, 3:4, :]
            iw = jnp.maximum(jnp.minimum(cx2, kx2) - jnp.maximum(cx1, kx1), 0.0)
            ih = jnp.maximum(jnp.minimum(cy2, ky2) - jnp.maximum(cy1, ky1), 0.0)
            inter = iw * ih
            union = carea + ka_ref[b:b + 1, :] - inter
            iou = inter / jnp.maximum(union, 1e-9)
            occ = slot < cs[b]
            sup = jnp.any(occ & (iou > _IOU_T))
            keep = act & jnp.logical_not(sup)
            wr = keep & (slot == cs[b])
            out_s_ref[b:b + 1, :] = jnp.where(wr, m, out_s_ref[b:b + 1, :])
            out_b_ref[b, 0:1, :] = jnp.where(wr, gx1, out_b_ref[b, 0:1, :])
            out_b_ref[b, 1:2, :] = jnp.where(wr, gy1, out_b_ref[b, 1:2, :])
            out_b_ref[b, 2:3, :] = jnp.where(wr, gx2, out_b_ref[b, 2:3, :])
            out_b_ref[b, 3:4, :] = jnp.where(wr, gy2, out_b_ref[b, 3:4, :])
            out_c_ref[b:b + 1, :] = jnp.where(wr, ccls, out_c_ref[b:b + 1, :])
            kb_ref[b, 0:1, :] = jnp.where(wr, cx1, kb_ref[b, 0:1, :])
            kb_ref[b, 1:2, :] = jnp.where(wr, cy1, kb_ref[b, 1:2, :])
            kb_ref[b, 2:3, :] = jnp.where(wr, cx2, kb_ref[b, 2:3, :])
            kb_ref[b, 3:4, :] = jnp.where(wr, cy2, kb_ref[b, 3:4, :])
            ka_ref[b:b + 1, :] = jnp.where(wr, carea, ka_ref[b:b + 1, :])
            newchunk = jnp.where(oh & act, -1.0, chunk)
            work_ref[pl.ds(r, 1), :] = newchunk
            cm_ref[pl.ds(r, 1), :] = jnp.full((1, 1), jnp.max(newchunk))
            cnew = cs[b] + keep.astype(jnp.int32)
            ds[b] = ds[b] | (m <= _SCORE_T) | (cnew >= _MAXDET)
            cs[b] = cnew
        return (t + 1,) + tuple(ds) + tuple(cs)

    def cond(carry):
        alldone = carry[1] & carry[2] & carry[3] & carry[4]
        return jnp.logical_not(alldone) & (carry[0] < _NPAD + 8)

    f = jnp.bool_(False)
    z = jnp.int32(0)
    final = jax.lax.while_loop(cond, iter_body, (z, f, f, f, f, z, z, z, z))

    rown = jax.lax.broadcasted_iota(jnp.int32, (8, 128), 0)
    coln = jax.lax.broadcasted_iota(jnp.int32, (8, 128), 1)
    nvec = jnp.zeros((8, 128), jnp.int32)
    for b in range(_B):
        nvec = jnp.where((rown == b) & (coln == 0), final[5 + b], nvec)
    out_n_ref[...] = nvec


def _nms_call(scores_p, x1, y1, x2, y2, cls_p):
    return pl.pallas_call(
        _nms_kernel,
        out_shape=[
            jax.ShapeDtypeStruct((_B, _OUTW), jnp.float32),
            jax.ShapeDtypeStruct((_B, 4, _OUTW), jnp.float32),
            jax.ShapeDtypeStruct((_B, _OUTW), jnp.int32),
            jax.ShapeDtypeStruct((8, 128), jnp.int32),
        ],
        scratch_shapes=[
            pltpu.VMEM((_ROWS, _CHUNK), jnp.float32),
            pltpu.VMEM((_ROWS, 1), jnp.float32),
            pltpu.VMEM((_B, 4, _OUTW), jnp.float32),
            pltpu.VMEM((_B, _OUTW), jnp.float32),
        ],
    )(scores_p, x1, y1, x2, y2, cls_p)


def kernel(scores, boxes, classes):
    B_, N_ = scores.shape
    pad = _NPAD - N_
    shp = (_ROWS, _CHUNK)
    scores_p = jnp.pad(scores, ((0, 0), (0, pad)), constant_values=-1.0).reshape(shp)
    x1 = jnp.pad(boxes[..., 0], ((0, 0), (0, pad))).reshape(shp)
    y1 = jnp.pad(boxes[..., 1], ((0, 0), (0, pad))).reshape(shp)
    x2 = jnp.pad(boxes[..., 2], ((0, 0), (0, pad))).reshape(shp)
    y2 = jnp.pad(boxes[..., 3], ((0, 0), (0, pad))).reshape(shp)
    cls_p = jnp.pad(classes.astype(jnp.int32), ((0, 0), (0, pad))).reshape(shp)
    out_s, out_b, out_c, out_n = _nms_call(scores_p, x1, y1, x2, y2, cls_p)
    dummy = jnp.full((B_, _MAXDET), -1, jnp.int32)
    return (dummy,
            out_s[:, :_MAXDET],
            jnp.transpose(out_b, (0, 2, 1))[:, :_MAXDET, :],
            out_c[:, :_MAXDET],
            out_n[:B_, 0])


# speedup vs baseline: 4.2706x; 4.2706x over previous
import jax
import jax.numpy as jnp
from jax.experimental import pallas as pl
from jax.experimental.pallas import tpu as pltpu

_IOU_T = 0.5
_SCORE_T = 0.8
_MAXDET = 300
_OUTW = 384
_B = 4
_NPAD = 5120


def _nms_kernel(scores_ref, x1_ref, y1_ref, x2_ref, y2_ref, cls_ref,
                out_s_ref, obx1_ref, oby1_ref, obx2_ref, oby2_ref,
                out_c_ref, out_n_ref,
                work_ref, kx1_ref, ky1_ref, kx2_ref, ky2_ref, ka_ref,
                cnt_ref, done_ref):
    lane = jax.lax.broadcasted_iota(jnp.int32, (_B, _NPAD), 1)
    slot = jax.lax.broadcasted_iota(jnp.int32, (_B, _OUTW), 1)

    work_ref[...] = scores_ref[...]
    out_s_ref[...] = jnp.zeros_like(out_s_ref)
    obx1_ref[...] = jnp.zeros_like(obx1_ref)
    oby1_ref[...] = jnp.zeros_like(oby1_ref)
    obx2_ref[...] = jnp.zeros_like(obx2_ref)
    oby2_ref[...] = jnp.zeros_like(oby2_ref)
    out_c_ref[...] = jnp.zeros_like(out_c_ref)
    kx1_ref[...] = jnp.zeros_like(kx1_ref)
    ky1_ref[...] = jnp.zeros_like(ky1_ref)
    kx2_ref[...] = jnp.zeros_like(kx2_ref)
    ky2_ref[...] = jnp.zeros_like(ky2_ref)
    ka_ref[...] = jnp.zeros_like(ka_ref)
    cnt_ref[...] = jnp.zeros_like(cnt_ref)
    done_ref[...] = jnp.zeros_like(done_ref)

    valid = scores_ref[...] > _SCORE_T
    mc = jnp.float32(-jnp.inf)
    for pref in (x1_ref, y1_ref, x2_ref, y2_ref):
        mc = jnp.maximum(mc, jnp.max(jnp.where(valid, pref[...], -jnp.inf)))
    mc = mc + 1.0
    bidx = jax.lax.broadcasted_iota(jnp.int32, (_B, 128), 0)
    off4 = (bidx.astype(jnp.float32) * mc)[:, 0:1]

    def iter_body(carry):
        t = carry[0]
        done = done_ref[:, 0:1] != 0
        cnt = cnt_ref[:, 0:1]
        work = work_ref[...]
        m4 = jnp.max(work, axis=1, keepdims=True)
        idx4 = jnp.min(jnp.where(work == m4, lane, _NPAD), axis=1,
                       keepdims=True)
        oh = lane == idx4
        act = (m4 > _SCORE_T) & jnp.logical_not(done)
        gx1 = jnp.sum(jnp.where(oh, x1_ref[...], 0.0), axis=1, keepdims=True)
        gy1 = jnp.sum(jnp.where(oh, y1_ref[...], 0.0), axis=1, keepdims=True)
        gx2 = jnp.sum(jnp.where(oh, x2_ref[...], 0.0), axis=1, keepdims=True)
        gy2 = jnp.sum(jnp.where(oh, y2_ref[...], 0.0), axis=1, keepdims=True)
        ccls = jnp.sum(jnp.where(oh, cls_ref[...], 0), axis=1, keepdims=True)
        cx1 = gx1 + off4
        cy1 = gy1 + off4
        cx2 = gx2 + off4
        cy2 = gy2 + off4
        carea = jnp.maximum(cx2 - cx1, 0.0) * jnp.maximum(cy2 - cy1, 0.0)
        iw = jnp.maximum(jnp.minimum(cx2, kx2_ref[...]) -
                         jnp.maximum(cx1, kx1_ref[...]), 0.0)
        ih = jnp.maximum(jnp.minimum(cy2, ky2_ref[...]) -
                         jnp.maximum(cy1, ky1_ref[...]), 0.0)
        inter = iw * ih
        union = carea + ka_ref[...] - inter
        iou = inter / jnp.maximum(union, 1e-9)
        occ = slot < cnt
        sup = jnp.any(occ & (iou > _IOU_T), axis=1, keepdims=True)
        keep = act & jnp.logical_not(sup)
        wr = keep & (slot == cnt)
        out_s_ref[...] = jnp.where(wr, m4, out_s_ref[...])
        obx1_ref[...] = jnp.where(wr, gx1, obx1_ref[...])
        oby1_ref[...] = jnp.where(wr, gy1, oby1_ref[...])
        obx2_ref[...] = jnp.where(wr, gx2, obx2_ref[...])
        oby2_ref[...] = jnp.where(wr, gy2, oby2_ref[...])
        out_c_ref[...] = jnp.where(wr, ccls, out_c_ref[...])
        kx1_ref[...] = jnp.where(wr, cx1, kx1_ref[...])
        ky1_ref[...] = jnp.where(wr, cy1, ky1_ref[...])
        kx2_ref[...] = jnp.where(wr, cx2, kx2_ref[...])
        ky2_ref[...] = jnp.where(wr, cy2, ky2_ref[...])
        ka_ref[...] = jnp.where(wr, carea, ka_ref[...])
        work_ref[...] = jnp.where(oh & act, -1.0, work)
        cnew = cnt + keep.astype(jnp.int32)
        dnew = done | (m4 <= _SCORE_T) | (cnew >= _MAXDET)
        cnt_ref[...] = jnp.broadcast_to(cnew, (_B, 128))
        done_ref[...] = jnp.broadcast_to(dnew.astype(jnp.int32), (_B, 128))
        return (t + 1, jnp.all(dnew))

    def cond(carry):
        return jnp.logical_not(carry[1]) & (carry[0] < _NPAD + 8)

    jax.lax.while_loop(cond, iter_body, (jnp.int32(0), jnp.bool_(False)))

    cnt = cnt_ref[:, 0:1]
    coln = jax.lax.broadcasted_iota(jnp.int32, (8, 128), 1)
    cpad = jnp.concatenate([cnt, jnp.zeros((8 - _B, 1), jnp.int32)], axis=0)
    out_n_ref[...] = jnp.where(coln == 0, cpad, 0)


def _nms_call(scores_p, x1, y1, x2, y2, cls_p):
    return pl.pallas_call(
        _nms_kernel,
        out_shape=[
            jax.ShapeDtypeStruct((_B, _OUTW), jnp.float32),
            jax.ShapeDtypeStruct((_B, _OUTW), jnp.float32),
            jax.ShapeDtypeStruct((_B, _OUTW), jnp.float32),
            jax.ShapeDtypeStruct((_B, _OUTW), jnp.float32),
            jax.ShapeDtypeStruct((_B, _OUTW), jnp.float32),
            jax.ShapeDtypeStruct((_B, _OUTW), jnp.int32),
            jax.ShapeDtypeStruct((8, 128), jnp.int32),
        ],
        scratch_shapes=[
            pltpu.VMEM((_B, _NPAD), jnp.float32),
            pltpu.VMEM((_B, _OUTW), jnp.float32),
            pltpu.VMEM((_B, _OUTW), jnp.float32),
            pltpu.VMEM((_B, _OUTW), jnp.float32),
            pltpu.VMEM((_B, _OUTW), jnp.float32),
            pltpu.VMEM((_B, _OUTW), jnp.float32),
            pltpu.VMEM((_B, 128), jnp.int32),
            pltpu.VMEM((_B, 128), jnp.int32),
        ],
    )(scores_p, x1, y1, x2, y2, cls_p)


def kernel(scores, boxes, classes):
    B_, N_ = scores.shape
    pad = _NPAD - N_
    scores_p = jnp.pad(scores, ((0, 0), (0, pad)), constant_values=-1.0)
    x1 = jnp.pad(boxes[..., 0], ((0, 0), (0, pad)))
    y1 = jnp.pad(boxes[..., 1], ((0, 0), (0, pad)))
    x2 = jnp.pad(boxes[..., 2], ((0, 0), (0, pad)))
    y2 = jnp.pad(boxes[..., 3], ((0, 0), (0, pad)))
    cls_p = jnp.pad(classes.astype(jnp.int32), ((0, 0), (0, pad)))
    out_s, ox1, oy1, ox2, oy2, out_c, out_n = _nms_call(
        scores_p, x1, y1, x2, y2, cls_p)
    dummy = jnp.full((B_, _MAXDET), -1, jnp.int32)
    boxes_o = jnp.stack([ox1, oy1, ox2, oy2], axis=-1)[:, :_MAXDET, :]
    return (dummy,
            out_s[:, :_MAXDET],
            boxes_o,
            out_c[:, :_MAXDET],
            out_n[:B_, 0])


# speedup vs baseline: 5.4331x; 1.2722x over previous
import jax
import jax.numpy as jnp
from jax import lax
from jax.experimental import pallas as pl
from jax.experimental.pallas import tpu as pltpu
from jax.experimental.pallas import tpu_sc as plsc

_IOU_T = 0.5
_SCORE_T = 0.8
_MAXDET = 300
_OUTW = 384
_B = 4
_NPAD = 5120
_L = 16
_NCH = _NPAD // _L
_NCHV = _NCH // _L
_KCAP = 304
_KV = _KCAP // _L
_OV = _OUTW // _L

_NEG = float("-inf")

_GDN = lax.GatherDimensionNumbers(offset_dims=(), collapsed_slice_dims=(0,),
                                  start_index_map=(0,))


def _sc_nms_body(scores_hbm, x1_hbm, y1_hbm, x2_hbm, y2_hbm, cls_hbm,
                 out_s, out_x1, out_y1, out_x2, out_y2, out_c, out_n,
                 sraw, x1r, y1r, x2r, y2r, clsr,
                 kx1, ky1, kx2, ky2, ka,
                 os_, ox1, oy1, ox2, oy2, oc, onum,
                 cmv, stt, mcsh, mcbuf):
    c = lax.axis_index("c")
    s = lax.axis_index("s")
    active = (c == 0) & (s < _B)
    bb = jnp.minimum(s, _B - 1)
    iot = lax.iota(jnp.int32, _L)

    def fsplat(v):
        return jnp.full((_L,), v, jnp.float32)

    def isplat(v):
        return jnp.full((_L,), v, jnp.int32)

    def bfly(x, op):
        for d in (8, 4, 2, 1):
            g = lax.gather(x, (iot ^ d).reshape(_L, 1), _GDN, (1,),
                           mode=lax.GatherScatterMode.PROMISE_IN_BOUNDS)
            x = op(x, g)
        return x

    negv = fsplat(_NEG)
    thrv = fsplat(_SCORE_T)
    zerov = fsplat(0.0)
    zeroiv = isplat(0)

    @pl.when(active)
    def _stage():
        pltpu.sync_copy(scores_hbm.at[bb], sraw)
        pltpu.sync_copy(x1_hbm.at[bb], x1r)
        pltpu.sync_copy(y1_hbm.at[bb], y1r)
        pltpu.sync_copy(x2_hbm.at[bb], x2r)
        pltpu.sync_copy(y2_hbm.at[bb], y2r)
        pltpu.sync_copy(cls_hbm.at[bb], clsr)

    def _maxbody(i, acc):
        sv = sraw[pl.ds(i * _L, _L)]
        msk = sv > thrv
        for pr in (x1r, y1r, x2r, y2r):
            acc = jnp.maximum(acc, jnp.where(msk, pr[pl.ds(i * _L, _L)], negv))
        return acc

    mymaxv = bfly(lax.fori_loop(0, _NCH, _maxbody, negv), jnp.maximum)
    mygated = jnp.where(active, mymaxv[0], jnp.float32(_NEG))
    mcbuf[0, :] = fsplat(mygated)
    pltpu.sync_copy(mcbuf.at[0], mcsh.at[s])
    plsc.subcore_barrier()
    pltpu.sync_copy(mcsh, mcbuf)
    mcacc = negv
    for i in range(_L):
        mcacc = jnp.maximum(mcacc, mcbuf[i, :])
    mc = bfly(mcacc, jnp.maximum)[0] + 1.0
    off = lax.convert_element_type(bb, jnp.float32) * mc
    offv = fsplat(off)

    @pl.when(active)
    def _init():
        def _cmbody(i, _):
            mv = bfly(sraw[pl.ds(i * _L, _L)], jnp.maximum)
            vb = (i // _L) * _L
            v = cmv[pl.ds(vb, _L)]
            cmv[pl.ds(vb, _L)] = jnp.where(iot == isplat(i - vb), mv, v)
            return 0

        lax.fori_loop(0, _NCH, _cmbody, 0)
        for k in range(_KV):
            kx1[pl.ds(k * _L, _L)] = zerov
            ky1[pl.ds(k * _L, _L)] = zerov
            kx2[pl.ds(k * _L, _L)] = zerov
            ky2[pl.ds(k * _L, _L)] = zerov
            ka[pl.ds(k * _L, _L)] = zerov
        for k in range(_OV):
            os_[pl.ds(k * _L, _L)] = zerov
            ox1[pl.ds(k * _L, _L)] = zerov
            oy1[pl.ds(k * _L, _L)] = zerov
            ox2[pl.ds(k * _L, _L)] = zerov
            oy2[pl.ds(k * _L, _L)] = zerov
            oc[pl.ds(k * _L, _L)] = zeroiv

    stt[...] = jnp.where(iot == isplat(1),
                         isplat(jnp.where(active, jnp.int32(0), jnp.int32(1))),
                         zeroiv)

    def selbody(i, _):
        st = stt[pl.ds(0, _L)]
        cnt = st[0]
        done = st[1] > 0

        @pl.when(jnp.logical_not(done))
        def _step():
            acc = cmv[pl.ds(0, _L)]
            for j in range(1, _NCHV):
                acc = jnp.maximum(acc, cmv[pl.ds(j * _L, _L)])
            gmv = bfly(acc, jnp.maximum)
            gm = gmv[0]
            cacc = isplat(_NCH)
            for j in range(_NCHV):
                v = cmv[pl.ds(j * _L, _L)]
                cacc = jnp.minimum(
                    cacc, jnp.where(v == gmv, iot + isplat(j * _L), isplat(_NCH)))
            cidx = bfly(cacc, jnp.minimum)[0]
            base = cidx * _L
            sv = sraw[pl.ds(base, _L)]
            lidxv = bfly(jnp.where(sv == gmv, iot, isplat(_L)), jnp.minimum)
            oh = iot == lidxv
            gx1v = bfly(jnp.where(oh, x1r[pl.ds(base, _L)], zerov), jnp.add)
            gy1v = bfly(jnp.where(oh, y1r[pl.ds(base, _L)], zerov), jnp.add)
            gx2v = bfly(jnp.where(oh, x2r[pl.ds(base, _L)], zerov), jnp.add)
            gy2v = bfly(jnp.where(oh, y2r[pl.ds(base, _L)], zerov), jnp.add)
            cclsv = bfly(jnp.where(oh, clsr[pl.ds(base, _L)], zeroiv), jnp.add)
            cx1v = gx1v + offv
            cy1v = gy1v + offv
            cx2v = gx2v + offv
            cy2v = gy2v + offv
            cav = (jnp.maximum(cx2v - cx1v, zerov) *
                   jnp.maximum(cy2v - cy1v, zerov))
            supv = zerov
            for k in range(_KV):
                k1 = kx1[pl.ds(k * _L, _L)]
                l1 = ky1[pl.ds(k * _L, _L)]
                k2 = kx2[pl.ds(k * _L, _L)]
                l2 = ky2[pl.ds(k * _L, _L)]
                kav = ka[pl.ds(k * _L, _L)]
                iw = jnp.maximum(jnp.minimum(cx2v, k2) - jnp.maximum(cx1v, k1), zerov)
                ih = jnp.maximum(jnp.minimum(cy2v, l2) - jnp.maximum(cy1v, l1), zerov)
                inter = iw * ih
                union = cav + kav - inter
                iou = inter / jnp.maximum(union, fsplat(1e-9))
                supv = jnp.maximum(supv, iou)
            sup = bfly(supv, jnp.maximum)[0] > _IOU_T
            act = gm > _SCORE_T
            keep = act & jnp.logical_not(sup)
            wb = (cnt // _L) * _L
            wsl = jnp.where(keep, cnt - wb, jnp.int32(-1))
            ohw = iot == isplat(wsl)
            os_[pl.ds(wb, _L)] = jnp.where(ohw, gmv, os_[pl.ds(wb, _L)])
            ox1[pl.ds(wb, _L)] = jnp.where(ohw, gx1v, ox1[pl.ds(wb, _L)])
            oy1[pl.ds(wb, _L)] = jnp.where(ohw, gy1v, oy1[pl.ds(wb, _L)])
            ox2[pl.ds(wb, _L)] = jnp.where(ohw, gx2v, ox2[pl.ds(wb, _L)])
            oy2[pl.ds(wb, _L)] = jnp.where(ohw, gy2v, oy2[pl.ds(wb, _L)])
            oc[pl.ds(wb, _L)] = jnp.where(ohw, cclsv, oc[pl.ds(wb, _L)])
            kx1[pl.ds(wb, _L)] = jnp.where(ohw, cx1v, kx1[pl.ds(wb, _L)])
            ky1[pl.ds(wb, _L)] = jnp.where(ohw, cy1v, ky1[pl.ds(wb, _L)])
            kx2[pl.ds(wb, _L)] = jnp.where(ohw, cx2v, kx2[pl.ds(wb, _L)])
            ky2[pl.ds(wb, _L)] = jnp.where(ohw, cy2v, ky2[pl.ds(wb, _L)])
            ka[pl.ds(wb, _L)] = jnp.where(ohw, cav, ka[pl.ds(wb, _L)])
            rsl = jnp.where(act, lidxv[0], jnp.int32(-1))
            ohr = iot == isplat(rsl)
            newsv = jnp.where(ohr, negv, sv)
            sraw[pl.ds(base, _L)] = newsv
            newmv = bfly(newsv, jnp.maximum)
            cb = (cidx // _L) * _L
            csl = jnp.where(act, cidx - cb, jnp.int32(-1))
            ohc = iot == isplat(csl)
            cmv[pl.ds(cb, _L)] = jnp.where(ohc, newmv, cmv[pl.ds(cb, _L)])
            cnt2 = jnp.where(keep, cnt + 1, cnt)
            done2 = (gm <= _SCORE_T) | (cnt2 >= _MAXDET)
            d2i = jnp.where(done2, jnp.int32(1), jnp.int32(0))
            st2 = jnp.where(iot == isplat(0), isplat(cnt2),
                            jnp.where(iot == isplat(1), isplat(d2i), st))
            stt[pl.ds(0, _L)] = st2

        return 0

    lax.fori_loop(0, _NPAD + 2, selbody, 0)
    onum[...] = isplat(stt[pl.ds(0, _L)][0])

    @pl.when(active)
    def _writeback():
        pltpu.sync_copy(os_, out_s.at[bb])
        pltpu.sync_copy(ox1, out_x1.at[bb])
        pltpu.sync_copy(oy1, out_y1.at[bb])
        pltpu.sync_copy(ox2, out_x2.at[bb])
        pltpu.sync_copy(oy2, out_y2.at[bb])
        pltpu.sync_copy(oc, out_c.at[bb])
        pltpu.sync_copy(onum, out_n.at[bb])


def _nms_call(scores_p, x1, y1, x2, y2, cls_p):
    mesh = plsc.VectorSubcoreMesh(core_axis_name="c", subcore_axis_name="s")
    fn = pl.kernel(
        _sc_nms_body,
        out_type=[
            jax.ShapeDtypeStruct((_B, _OUTW), jnp.float32),
            jax.ShapeDtypeStruct((_B, _OUTW), jnp.float32),
            jax.ShapeDtypeStruct((_B, _OUTW), jnp.float32),
            jax.ShapeDtypeStruct((_B, _OUTW), jnp.float32),
            jax.ShapeDtypeStruct((_B, _OUTW), jnp.float32),
            jax.ShapeDtypeStruct((_B, _OUTW), jnp.int32),
            jax.ShapeDtypeStruct((_B, _L), jnp.int32),
        ],
        mesh=mesh,
        scratch_types=[
            pltpu.VMEM((_NPAD,), jnp.float32),
            pltpu.VMEM((_NPAD,), jnp.float32),
            pltpu.VMEM((_NPAD,), jnp.float32),
            pltpu.VMEM((_NPAD,), jnp.float32),
            pltpu.VMEM((_NPAD,), jnp.float32),
            pltpu.VMEM((_NPAD,), jnp.int32),
            pltpu.VMEM((_KCAP,), jnp.float32),
            pltpu.VMEM((_KCAP,), jnp.float32),
            pltpu.VMEM((_KCAP,), jnp.float32),
            pltpu.VMEM((_KCAP,), jnp.float32),
            pltpu.VMEM((_KCAP,), jnp.float32),
            pltpu.VMEM((_OUTW,), jnp.float32),
            pltpu.VMEM((_OUTW,), jnp.float32),
            pltpu.VMEM((_OUTW,), jnp.float32),
            pltpu.VMEM((_OUTW,), jnp.float32),
            pltpu.VMEM((_OUTW,), jnp.float32),
            pltpu.VMEM((_OUTW,), jnp.int32),
            pltpu.VMEM((_L,), jnp.int32),
            pltpu.VMEM((_NCH,), jnp.float32),
            pltpu.VMEM((_L,), jnp.int32),
            pltpu.VMEM_SHARED((_L, _L), jnp.float32),
            pltpu.VMEM((_L, _L), jnp.float32),
        ],
    )
    return fn(scores_p, x1, y1, x2, y2, cls_p)


def kernel(scores, boxes, classes):
    B_, N_ = scores.shape
    pad = _NPAD - N_
    scores_p = jnp.pad(scores, ((0, 0), (0, pad)), constant_values=-1.0)
    x1 = jnp.pad(boxes[..., 0], ((0, 0), (0, pad)))
    y1 = jnp.pad(boxes[..., 1], ((0, 0), (0, pad)))
    x2 = jnp.pad(boxes[..., 2], ((0, 0), (0, pad)))
    y2 = jnp.pad(boxes[..., 3], ((0, 0), (0, pad)))
    cls_p = jnp.pad(classes.astype(jnp.int32), ((0, 0), (0, pad)))
    out_s, ox1, oy1, ox2, oy2, out_c, out_n = _nms_call(
        scores_p, x1, y1, x2, y2, cls_p)
    dummy = jnp.full((B_, _MAXDET), -1, jnp.int32)
    boxes_o = jnp.stack([ox1, oy1, ox2, oy2], axis=-1)[:, :_MAXDET, :]
    return (dummy,
            out_s[:, :_MAXDET],
            boxes_o,
            out_c[:, :_MAXDET],
            out_n[:B_, 0])


# speedup vs baseline: 8.2719x; 1.5225x over previous
import jax
import jax.numpy as jnp
from jax import lax
from jax.experimental import pallas as pl
from jax.experimental.pallas import tpu as pltpu
from jax.experimental.pallas import tpu_sc as plsc

_IOU_T = 0.5
_SCORE_T = 0.8
_MAXDET = 300
_OUTW = 384
_B = 4
_NPAD = 5120
_L = 16
_NCH = _NPAD // _L
_NCHV = _NCH // _L
_KCAP = 304
_KV = _KCAP // _L
_OV = _OUTW // _L

_NEG = float("-inf")

_GDN = lax.GatherDimensionNumbers(offset_dims=(), collapsed_slice_dims=(0,),
                                  start_index_map=(0,))


def _sc_nms_body(scores_hbm, x1_hbm, y1_hbm, x2_hbm, y2_hbm, cls_hbm,
                 out_s, out_x1, out_y1, out_x2, out_y2, out_c, out_n,
                 sraw, x1r, y1r, x2r, y2r, clsr,
                 kx1, ky1, kx2, ky2, ka,
                 os_, ox1, oy1, ox2, oy2, oc, onum,
                 cmv, stt, mcsh, mcbuf):
    c = lax.axis_index("c")
    s = lax.axis_index("s")
    active = (c == 0) & (s < _B)
    bb = jnp.minimum(s, _B - 1)
    iot = lax.iota(jnp.int32, _L)

    def fsplat(v):
        return jnp.full((_L,), v, jnp.float32)

    def isplat(v):
        return jnp.full((_L,), v, jnp.int32)

    def bfly(x, op):
        for d in (8, 4, 2, 1):
            g = lax.gather(x, (iot ^ d).reshape(_L, 1), _GDN, (1,),
                           mode=lax.GatherScatterMode.PROMISE_IN_BOUNDS)
            x = op(x, g)
        return x

    negv = fsplat(_NEG)
    thrv = fsplat(_SCORE_T)
    zerov = fsplat(0.0)
    zeroiv = isplat(0)

    @pl.when(active)
    def _stage():
        pltpu.sync_copy(scores_hbm.at[bb], sraw)
        pltpu.sync_copy(x1_hbm.at[bb], x1r)
        pltpu.sync_copy(y1_hbm.at[bb], y1r)
        pltpu.sync_copy(x2_hbm.at[bb], x2r)
        pltpu.sync_copy(y2_hbm.at[bb], y2r)
        pltpu.sync_copy(cls_hbm.at[bb], clsr)

    def _maxbody(i, acc):
        sv = sraw[pl.ds(i * _L, _L)]
        msk = sv > thrv
        for pr in (x1r, y1r, x2r, y2r):
            acc = jnp.maximum(acc, jnp.where(msk, pr[pl.ds(i * _L, _L)], negv))
        return acc

    mymaxv = bfly(lax.fori_loop(0, _NCH, _maxbody, negv), jnp.maximum)
    mygated = jnp.where(active, mymaxv[0], jnp.float32(_NEG))
    mcbuf[0, :] = fsplat(mygated)
    pltpu.sync_copy(mcbuf.at[0], mcsh.at[s])
    plsc.subcore_barrier()
    pltpu.sync_copy(mcsh, mcbuf)
    mcacc = negv
    for i in range(_L):
        mcacc = jnp.maximum(mcacc, mcbuf[i, :])
    mc = bfly(mcacc, jnp.maximum)[0] + 1.0
    off = lax.convert_element_type(bb, jnp.float32) * mc
    offv = fsplat(off)

    @pl.when(active)
    def _init():
        def _cmbody(i, _):
            mv = bfly(sraw[pl.ds(i * _L, _L)], jnp.maximum)
            vb = (i // _L) * _L
            v = cmv[pl.ds(vb, _L)]
            cmv[pl.ds(vb, _L)] = jnp.where(iot == isplat(i - vb), mv, v)
            return 0

        lax.fori_loop(0, _NCH, _cmbody, 0)
        for k in range(_KV):
            kx1[pl.ds(k * _L, _L)] = zerov
            ky1[pl.ds(k * _L, _L)] = zerov
            kx2[pl.ds(k * _L, _L)] = zerov
            ky2[pl.ds(k * _L, _L)] = zerov
            ka[pl.ds(k * _L, _L)] = zerov
        for k in range(_OV):
            os_[pl.ds(k * _L, _L)] = zerov
            ox1[pl.ds(k * _L, _L)] = zerov
            oy1[pl.ds(k * _L, _L)] = zerov
            ox2[pl.ds(k * _L, _L)] = zerov
            oy2[pl.ds(k * _L, _L)] = zerov
            oc[pl.ds(k * _L, _L)] = zeroiv

    stt[...] = jnp.where(iot == isplat(1),
                         isplat(jnp.where(active, jnp.int32(0), jnp.int32(1))),
                         zeroiv)
    _BLK = 8

    def selbody(i, _):
        st = stt[pl.ds(0, _L)]
        done0 = st[1] > 0

        @pl.when(jnp.logical_not(done0))
        def _block():
            cnt = st[0]
            gm = jnp.float32(0.0)
            for _k in range(_BLK):
                acc = cmv[pl.ds(0, _L)]
                for j in range(1, _NCHV):
                    acc = jnp.maximum(acc, cmv[pl.ds(j * _L, _L)])
                gmv = bfly(acc, jnp.maximum)
                gm = gmv[0]
                cacc = isplat(_NCH)
                for j in range(_NCHV):
                    v = cmv[pl.ds(j * _L, _L)]
                    cacc = jnp.minimum(
                        cacc, jnp.where(v == gmv, iot + isplat(j * _L), isplat(_NCH)))
                cidx = bfly(cacc, jnp.minimum)[0]
                base = cidx * _L
                sv = sraw[pl.ds(base, _L)]
                lidxv = bfly(jnp.where(sv == gmv, iot, isplat(_L)), jnp.minimum)
                oh = iot == lidxv
                gx1v = bfly(jnp.where(oh, x1r[pl.ds(base, _L)], zerov), jnp.add)
                gy1v = bfly(jnp.where(oh, y1r[pl.ds(base, _L)], zerov), jnp.add)
                gx2v = bfly(jnp.where(oh, x2r[pl.ds(base, _L)], zerov), jnp.add)
                gy2v = bfly(jnp.where(oh, y2r[pl.ds(base, _L)], zerov), jnp.add)
                cclsv = bfly(jnp.where(oh, clsr[pl.ds(base, _L)], zeroiv), jnp.add)
                cx1v = gx1v + offv
                cy1v = gy1v + offv
                cx2v = gx2v + offv
                cy2v = gy2v + offv
                cav = (jnp.maximum(cx2v - cx1v, zerov) *
                       jnp.maximum(cy2v - cy1v, zerov))
                supv = zerov
                for k in range(_KV):
                    k1 = kx1[pl.ds(k * _L, _L)]
                    l1 = ky1[pl.ds(k * _L, _L)]
                    k2 = kx2[pl.ds(k * _L, _L)]
                    l2 = ky2[pl.ds(k * _L, _L)]
                    kav = ka[pl.ds(k * _L, _L)]
                    iw = jnp.maximum(jnp.minimum(cx2v, k2) - jnp.maximum(cx1v, k1), zerov)
                    ih = jnp.maximum(jnp.minimum(cy2v, l2) - jnp.maximum(cy1v, l1), zerov)
                    inter = iw * ih
                    union = cav + kav - inter
                    iou = inter / jnp.maximum(union, fsplat(1e-9))
                    supv = jnp.maximum(supv, iou)
                sup = bfly(supv, jnp.maximum)[0] > _IOU_T
                act = (gm > _SCORE_T) & (cnt < _MAXDET)
                keep = act & jnp.logical_not(sup)
                wb = (cnt // _L) * _L
                wsl = jnp.where(keep, cnt - wb, jnp.int32(-1))
                ohw = iot == isplat(wsl)
                os_[pl.ds(wb, _L)] = jnp.where(ohw, gmv, os_[pl.ds(wb, _L)])
                ox1[pl.ds(wb, _L)] = jnp.where(ohw, gx1v, ox1[pl.ds(wb, _L)])
                oy1[pl.ds(wb, _L)] = jnp.where(ohw, gy1v, oy1[pl.ds(wb, _L)])
                ox2[pl.ds(wb, _L)] = jnp.where(ohw, gx2v, ox2[pl.ds(wb, _L)])
                oy2[pl.ds(wb, _L)] = jnp.where(ohw, gy2v, oy2[pl.ds(wb, _L)])
                oc[pl.ds(wb, _L)] = jnp.where(ohw, cclsv, oc[pl.ds(wb, _L)])
                kx1[pl.ds(wb, _L)] = jnp.where(ohw, cx1v, kx1[pl.ds(wb, _L)])
                ky1[pl.ds(wb, _L)] = jnp.where(ohw, cy1v, ky1[pl.ds(wb, _L)])
                kx2[pl.ds(wb, _L)] = jnp.where(ohw, cx2v, kx2[pl.ds(wb, _L)])
                ky2[pl.ds(wb, _L)] = jnp.where(ohw, cy2v, ky2[pl.ds(wb, _L)])
                ka[pl.ds(wb, _L)] = jnp.where(ohw, cav, ka[pl.ds(wb, _L)])
                rsl = jnp.where(act, lidxv[0], jnp.int32(-1))
                ohr = iot == isplat(rsl)
                newsv = jnp.where(ohr, negv, sv)
                sraw[pl.ds(base, _L)] = newsv
                newmv = bfly(newsv, jnp.maximum)
                cb = (cidx // _L) * _L
                csl = jnp.where(act, cidx - cb, jnp.int32(-1))
                ohc = iot == isplat(csl)
                cmv[pl.ds(cb, _L)] = jnp.where(ohc, newmv, cmv[pl.ds(cb, _L)])
                cnt = jnp.where(keep, cnt + 1, cnt)
            done2 = (gm <= _SCORE_T) | (cnt >= _MAXDET)
            d2i = jnp.where(done2, jnp.int32(1), jnp.int32(0))
            st2 = jnp.where(iot == isplat(0), isplat(cnt),
                            jnp.where(iot == isplat(1), isplat(d2i), st))
            stt[pl.ds(0, _L)] = st2

        return 0

    lax.fori_loop(0, (_NPAD + 2 + _BLK - 1) // _BLK, selbody, 0)
    onum[...] = isplat(stt[pl.ds(0, _L)][0])

    @pl.when(active)
    def _writeback():
        pltpu.sync_copy(os_, out_s.at[bb])
        pltpu.sync_copy(ox1, out_x1.at[bb])
        pltpu.sync_copy(oy1, out_y1.at[bb])
        pltpu.sync_copy(ox2, out_x2.at[bb])
        pltpu.sync_copy(oy2, out_y2.at[bb])
        pltpu.sync_copy(oc, out_c.at[bb])
        pltpu.sync_copy(onum, out_n.at[bb])


def _nms_call(scores_p, x1, y1, x2, y2, cls_p):
    mesh = plsc.VectorSubcoreMesh(core_axis_name="c", subcore_axis_name="s")
    fn = pl.kernel(
        _sc_nms_body,
        out_type=[
            jax.ShapeDtypeStruct((_B, _OUTW), jnp.float32),
            jax.ShapeDtypeStruct((_B, _OUTW), jnp.float32),
            jax.ShapeDtypeStruct((_B, _OUTW), jnp.float32),
            jax.ShapeDtypeStruct((_B, _OUTW), jnp.float32),
            jax.ShapeDtypeStruct((_B, _OUTW), jnp.float32),
            jax.ShapeDtypeStruct((_B, _OUTW), jnp.int32),
            jax.ShapeDtypeStruct((_B, _L), jnp.int32),
        ],
        mesh=mesh,
        scratch_types=[
            pltpu.VMEM((_NPAD,), jnp.float32),
            pltpu.VMEM((_NPAD,), jnp.float32),
            pltpu.VMEM((_NPAD,), jnp.float32),
            pltpu.VMEM((_NPAD,), jnp.float32),
            pltpu.VMEM((_NPAD,), jnp.float32),
            pltpu.VMEM((_NPAD,), jnp.int32),
            pltpu.VMEM((_KCAP,), jnp.float32),
            pltpu.VMEM((_KCAP,), jnp.float32),
            pltpu.VMEM((_KCAP,), jnp.float32),
            pltpu.VMEM((_KCAP,), jnp.float32),
            pltpu.VMEM((_KCAP,), jnp.float32),
            pltpu.VMEM((_OUTW,), jnp.float32),
            pltpu.VMEM((_OUTW,), jnp.float32),
            pltpu.VMEM((_OUTW,), jnp.float32),
            pltpu.VMEM((_OUTW,), jnp.float32),
            pltpu.VMEM((_OUTW,), jnp.float32),
            pltpu.VMEM((_OUTW,), jnp.int32),
            pltpu.VMEM((_L,), jnp.int32),
            pltpu.VMEM((_NCH,), jnp.float32),
            pltpu.VMEM((_L,), jnp.int32),
            pltpu.VMEM_SHARED((_L, _L), jnp.float32),
            pltpu.VMEM((_L, _L), jnp.float32),
        ],
    )
    return fn(scores_p, x1, y1, x2, y2, cls_p)


def kernel(scores, boxes, classes):
    B_, N_ = scores.shape
    pad = _NPAD - N_
    scores_p = jnp.pad(scores, ((0, 0), (0, pad)), constant_values=-1.0)
    x1 = jnp.pad(boxes[..., 0], ((0, 0), (0, pad)))
    y1 = jnp.pad(boxes[..., 1], ((0, 0), (0, pad)))
    x2 = jnp.pad(boxes[..., 2], ((0, 0), (0, pad)))
    y2 = jnp.pad(boxes[..., 3], ((0, 0), (0, pad)))
    cls_p = jnp.pad(classes.astype(jnp.int32), ((0, 0), (0, pad)))
    out_s, ox1, oy1, ox2, oy2, out_c, out_n = _nms_call(
        scores_p, x1, y1, x2, y2, cls_p)
    dummy = jnp.full((B_, _MAXDET), -1, jnp.int32)
    boxes_o = jnp.stack([ox1, oy1, ox2, oy2], axis=-1)[:, :_MAXDET, :]
    return (dummy,
            out_s[:, :_MAXDET],
            boxes_o,
            out_c[:, :_MAXDET],
            out_n[:B_, 0])


# speedup vs baseline: 8.7327x; 1.0557x over previous
import jax
import jax.numpy as jnp
from jax import lax
from jax.experimental import pallas as pl
from jax.experimental.pallas import tpu as pltpu
from jax.experimental.pallas import tpu_sc as plsc

_IOU_T = 0.5
_SCORE_T = 0.8
_MAXDET = 300
_OUTW = 384
_B = 4
_NPAD = 5120
_L = 16
_NCH = _NPAD // _L
_NCHV = _NCH // _L
_KCAP = 304
_KV = _KCAP // _L
_OV = _OUTW // _L

_NEG = float("-inf")

_GDN = lax.GatherDimensionNumbers(offset_dims=(), collapsed_slice_dims=(0,),
                                  start_index_map=(0,))


def _sc_nms_body(scores_hbm, x1_hbm, y1_hbm, x2_hbm, y2_hbm, cls_hbm,
                 out_s, out_x1, out_y1, out_x2, out_y2, out_c, out_n,
                 sraw, x1r, y1r, x2r, y2r, clsr,
                 kx1, ky1, kx2, ky2, ka,
                 os_, ox1, oy1, ox2, oy2, oc, onum,
                 cmv, stt, mcsh, mcbuf):
    c = lax.axis_index("c")
    s = lax.axis_index("s")
    active = (c == 0) & (s < _B)
    bb = jnp.minimum(s, _B - 1)
    iot = lax.iota(jnp.int32, _L)

    def fsplat(v):
        return jnp.full((_L,), v, jnp.float32)

    def isplat(v):
        return jnp.full((_L,), v, jnp.int32)

    def bfly(x, op):
        for d in (8, 4, 2, 1):
            g = lax.gather(x, (iot ^ d).reshape(_L, 1), _GDN, (1,),
                           mode=lax.GatherScatterMode.PROMISE_IN_BOUNDS)
            x = op(x, g)
        return x

    negv = fsplat(_NEG)
    thrv = fsplat(_SCORE_T)
    zerov = fsplat(0.0)
    zeroiv = isplat(0)

    @pl.when(active)
    def _stage():
        pltpu.sync_copy(scores_hbm.at[bb], sraw)
        pltpu.sync_copy(x1_hbm.at[bb], x1r)
        pltpu.sync_copy(y1_hbm.at[bb], y1r)
        pltpu.sync_copy(x2_hbm.at[bb], x2r)
        pltpu.sync_copy(y2_hbm.at[bb], y2r)
        pltpu.sync_copy(cls_hbm.at[bb], clsr)

    def _maxbody(i, acc):
        sv = sraw[pl.ds(i * _L, _L)]
        msk = sv > thrv
        for pr in (x1r, y1r, x2r, y2r):
            acc = jnp.maximum(acc, jnp.where(msk, pr[pl.ds(i * _L, _L)], negv))
        return acc

    mymaxv = bfly(lax.fori_loop(0, _NCH, _maxbody, negv), jnp.maximum)
    mygated = jnp.where(active, mymaxv[0], jnp.float32(_NEG))
    mcbuf[0, :] = fsplat(mygated)
    pltpu.sync_copy(mcbuf.at[0], mcsh.at[s])
    plsc.subcore_barrier()
    pltpu.sync_copy(mcsh, mcbuf)
    mcacc = negv
    for i in range(_L):
        mcacc = jnp.maximum(mcacc, mcbuf[i, :])
    mc = bfly(mcacc, jnp.maximum)[0] + 1.0
    off = lax.convert_element_type(bb, jnp.float32) * mc
    offv = fsplat(off)

    @pl.when(active)
    def _init():
        def _cmbody(i, _):
            mv = bfly(sraw[pl.ds(i * _L, _L)], jnp.maximum)
            vb = (i // _L) * _L
            v = cmv[pl.ds(vb, _L)]
            cmv[pl.ds(vb, _L)] = jnp.where(iot == isplat(i - vb), mv, v)
            return 0

        lax.fori_loop(0, _NCH, _cmbody, 0)
        for k in range(_KV):
            kx1[pl.ds(k * _L, _L)] = zerov
            ky1[pl.ds(k * _L, _L)] = zerov
            kx2[pl.ds(k * _L, _L)] = zerov
            ky2[pl.ds(k * _L, _L)] = zerov
            ka[pl.ds(k * _L, _L)] = zerov
        for k in range(_OV):
            os_[pl.ds(k * _L, _L)] = zerov
            ox1[pl.ds(k * _L, _L)] = zerov
            oy1[pl.ds(k * _L, _L)] = zerov
            ox2[pl.ds(k * _L, _L)] = zerov
            oy2[pl.ds(k * _L, _L)] = zerov
            oc[pl.ds(k * _L, _L)] = zeroiv

    stt[...] = jnp.where(iot == isplat(1),
                         isplat(jnp.where(active, jnp.int32(0), jnp.int32(1))),
                         zeroiv)
    _BLK = 8

    def selbody(i, _):
        st = stt[pl.ds(0, _L)]
        done0 = st[1] > 0

        @pl.when(jnp.logical_not(done0))
        def _block():
            cnt = st[0]
            gm = jnp.float32(0.0)
            for _k in range(_BLK):
                acc = cmv[pl.ds(0, _L)]
                for j in range(1, _NCHV):
                    acc = jnp.maximum(acc, cmv[pl.ds(j * _L, _L)])
                gmv = bfly(acc, jnp.maximum)
                gm = gmv[0]
                cacc = isplat(_NCH)
                for j in range(_NCHV):
                    v = cmv[pl.ds(j * _L, _L)]
                    cacc = jnp.minimum(
                        cacc, jnp.where(v == gmv, iot + isplat(j * _L), isplat(_NCH)))
                cidx = bfly(cacc, jnp.minimum)[0]
                base = cidx * _L
                sv = sraw[pl.ds(base, _L)]
                lidxv = bfly(jnp.where(sv == gmv, iot, isplat(_L)), jnp.minimum)
                oh = iot == lidxv
                lix = lidxv.reshape(_L, 1)
                pm = lax.GatherScatterMode.PROMISE_IN_BOUNDS
                gx1v = lax.gather(x1r[pl.ds(base, _L)], lix, _GDN, (1,), mode=pm)
                gy1v = lax.gather(y1r[pl.ds(base, _L)], lix, _GDN, (1,), mode=pm)
                gx2v = lax.gather(x2r[pl.ds(base, _L)], lix, _GDN, (1,), mode=pm)
                gy2v = lax.gather(y2r[pl.ds(base, _L)], lix, _GDN, (1,), mode=pm)
                cclsv = lax.gather(clsr[pl.ds(base, _L)], lix, _GDN, (1,), mode=pm)
                cx1v = gx1v + offv
                cy1v = gy1v + offv
                cx2v = gx2v + offv
                cy2v = gy2v + offv
                cav = (jnp.maximum(cx2v - cx1v, zerov) *
                       jnp.maximum(cy2v - cy1v, zerov))
                supv = zerov
                for k in range(_KV):
                    k1 = kx1[pl.ds(k * _L, _L)]
                    l1 = ky1[pl.ds(k * _L, _L)]
                    k2 = kx2[pl.ds(k * _L, _L)]
                    l2 = ky2[pl.ds(k * _L, _L)]
                    kav = ka[pl.ds(k * _L, _L)]
                    iw = jnp.maximum(jnp.minimum(cx2v, k2) - jnp.maximum(cx1v, k1), zerov)
                    ih = jnp.maximum(jnp.minimum(cy2v, l2) - jnp.maximum(cy1v, l1), zerov)
                    inter = iw * ih
                    union = cav + kav - inter
                    iou = inter / jnp.maximum(union, fsplat(1e-9))
                    supv = jnp.maximum(supv, iou)
                sup = bfly(supv, jnp.maximum)[0] > _IOU_T
                act = (gm > _SCORE_T) & (cnt < _MAXDET)
                keep = act & jnp.logical_not(sup)
                wb = (cnt // _L) * _L
                wsl = jnp.where(keep, cnt - wb, jnp.int32(-1))
                ohw = iot == isplat(wsl)
                os_[pl.ds(wb, _L)] = jnp.where(ohw, gmv, os_[pl.ds(wb, _L)])
                ox1[pl.ds(wb, _L)] = jnp.where(ohw, gx1v, ox1[pl.ds(wb, _L)])
                oy1[pl.ds(wb, _L)] = jnp.where(ohw, gy1v, oy1[pl.ds(wb, _L)])
                ox2[pl.ds(wb, _L)] = jnp.where(ohw, gx2v, ox2[pl.ds(wb, _L)])
                oy2[pl.ds(wb, _L)] = jnp.where(ohw, gy2v, oy2[pl.ds(wb, _L)])
                oc[pl.ds(wb, _L)] = jnp.where(ohw, cclsv, oc[pl.ds(wb, _L)])
                kx1[pl.ds(wb, _L)] = jnp.where(ohw, cx1v, kx1[pl.ds(wb, _L)])
                ky1[pl.ds(wb, _L)] = jnp.where(ohw, cy1v, ky1[pl.ds(wb, _L)])
                kx2[pl.ds(wb, _L)] = jnp.where(ohw, cx2v, kx2[pl.ds(wb, _L)])
                ky2[pl.ds(wb, _L)] = jnp.where(ohw, cy2v, ky2[pl.ds(wb, _L)])
                ka[pl.ds(wb, _L)] = jnp.where(ohw, cav, ka[pl.ds(wb, _L)])
                rsl = jnp.where(act, lidxv[0], jnp.int32(-1))
                ohr = iot == isplat(rsl)
                newsv = jnp.where(ohr, negv, sv)
                sraw[pl.ds(base, _L)] = newsv
                newmv = bfly(newsv, jnp.maximum)
                cb = (cidx // _L) * _L
                csl = jnp.where(act, cidx - cb, jnp.int32(-1))
                ohc = iot == isplat(csl)
                cmv[pl.ds(cb, _L)] = jnp.where(ohc, newmv, cmv[pl.ds(cb, _L)])
                cnt = jnp.where(keep, cnt + 1, cnt)
            done2 = (gm <= _SCORE_T) | (cnt >= _MAXDET)
            d2i = jnp.where(done2, jnp.int32(1), jnp.int32(0))
            st2 = jnp.where(iot == isplat(0), isplat(cnt),
                            jnp.where(iot == isplat(1), isplat(d2i), st))
            stt[pl.ds(0, _L)] = st2

        return 0

    lax.fori_loop(0, (_NPAD + 2 + _BLK - 1) // _BLK, selbody, 0)
    onum[...] = isplat(stt[pl.ds(0, _L)][0])

    @pl.when(active)
    def _writeback():
        pltpu.sync_copy(os_, out_s.at[bb])
        pltpu.sync_copy(ox1, out_x1.at[bb])
        pltpu.sync_copy(oy1, out_y1.at[bb])
        pltpu.sync_copy(ox2, out_x2.at[bb])
        pltpu.sync_copy(oy2, out_y2.at[bb])
        pltpu.sync_copy(oc, out_c.at[bb])
        pltpu.sync_copy(onum, out_n.at[bb])


def _nms_call(scores_p, x1, y1, x2, y2, cls_p):
    mesh = plsc.VectorSubcoreMesh(core_axis_name="c", subcore_axis_name="s")
    fn = pl.kernel(
        _sc_nms_body,
        out_type=[
            jax.ShapeDtypeStruct((_B, _OUTW), jnp.float32),
            jax.ShapeDtypeStruct((_B, _OUTW), jnp.float32),
            jax.ShapeDtypeStruct((_B, _OUTW), jnp.float32),
            jax.ShapeDtypeStruct((_B, _OUTW), jnp.float32),
            jax.ShapeDtypeStruct((_B, _OUTW), jnp.float32),
            jax.ShapeDtypeStruct((_B, _OUTW), jnp.int32),
            jax.ShapeDtypeStruct((_B, _L), jnp.int32),
        ],
        mesh=mesh,
        scratch_types=[
            pltpu.VMEM((_NPAD,), jnp.float32),
            pltpu.VMEM((_NPAD,), jnp.float32),
            pltpu.VMEM((_NPAD,), jnp.float32),
            pltpu.VMEM((_NPAD,), jnp.float32),
            pltpu.VMEM((_NPAD,), jnp.float32),
            pltpu.VMEM((_NPAD,), jnp.int32),
            pltpu.VMEM((_KCAP,), jnp.float32),
            pltpu.VMEM((_KCAP,), jnp.float32),
            pltpu.VMEM((_KCAP,), jnp.float32),
            pltpu.VMEM((_KCAP,), jnp.float32),
            pltpu.VMEM((_KCAP,), jnp.float32),
            pltpu.VMEM((_OUTW,), jnp.float32),
            pltpu.VMEM((_OUTW,), jnp.float32),
            pltpu.VMEM((_OUTW,), jnp.float32),
            pltpu.VMEM((_OUTW,), jnp.float32),
            pltpu.VMEM((_OUTW,), jnp.float32),
            pltpu.VMEM((_OUTW,), jnp.int32),
            pltpu.VMEM((_L,), jnp.int32),
            pltpu.VMEM((_NCH,), jnp.float32),
            pltpu.VMEM((_L,), jnp.int32),
            pltpu.VMEM_SHARED((_L, _L), jnp.float32),
            pltpu.VMEM((_L, _L), jnp.float32),
        ],
    )
    return fn(scores_p, x1, y1, x2, y2, cls_p)


def kernel(scores, boxes, classes):
    B_, N_ = scores.shape
    pad = _NPAD - N_
    scores_p = jnp.pad(scores, ((0, 0), (0, pad)), constant_values=-1.0)
    x1 = jnp.pad(boxes[..., 0], ((0, 0), (0, pad)))
    y1 = jnp.pad(boxes[..., 1], ((0, 0), (0, pad)))
    x2 = jnp.pad(boxes[..., 2], ((0, 0), (0, pad)))
    y2 = jnp.pad(boxes[..., 3], ((0, 0), (0, pad)))
    cls_p = jnp.pad(classes.astype(jnp.int32), ((0, 0), (0, pad)))
    out_s, ox1, oy1, ox2, oy2, out_c, out_n = _nms_call(
        scores_p, x1, y1, x2, y2, cls_p)
    dummy = jnp.full((B_, _MAXDET), -1, jnp.int32)
    boxes_o = jnp.stack([ox1, oy1, ox2, oy2], axis=-1)[:, :_MAXDET, :]
    return (dummy,
            out_s[:, :_MAXDET],
            boxes_o,
            out_c[:, :_MAXDET],
            out_n[:B_, 0])
